# baseline (speedup 1.0000x reference)
import math

import jax
import jax.numpy as jnp
from jax import lax
from jax.experimental import pallas as pl
from jax.experimental.pallas import tpu as pltpu
from jax.experimental.pallas import tpu_sc as plsc

MAX_ATOM = 100
N_CRYST = 128
ATOMS_PER = 16
N_ATOMS = N_CRYST * ATOMS_PER
MAX_NEI = 20
E = N_ATOMS * MAX_NEI
SPH = 128
HID = 256
NRBF = 128
NINT = 8
CUTOFF = 8.0

NC = 2
NS = 16
NW = NC * NS
L = 16
EPW = E // NW
CHUNK = 64
NCHUNK = EPW // CHUNK
NPT = N_ATOMS // NW
CAPT = 1536
BB = 64

_F32 = jnp.float32


def _pre_body(frac_ref, len_ref, ang_ref, z_ref, zproj_ref, emb_ref, ty_ref,
              wab_ref, cart_ref, x0_ref, ha_ref, hb_ref):
    ang = ang_ref[...] * (math.pi / 180.0)
    cosv = jnp.cos(ang)
    sing = jnp.sin(ang[:, 2:3])
    a = len_ref[:, 0:1]
    b = len_ref[:, 1:2]
    c = len_ref[:, 2:3]
    cx = cosv[:, 1:2]
    cy = (cosv[:, 0:1] - cosv[:, 1:2] * cosv[:, 2:3]) / sing
    cz = jnp.sqrt(jnp.clip(1.0 - cx * cx - cy * cy, 1e-8, None))
    def rep(v):
        return jnp.broadcast_to(v[:, None, :], (N_CRYST, ATOMS_PER, 1)
                                ).reshape(N_ATOMS, 1)
    f = frac_ref[...]
    f0 = f[:, 0:1]
    f1 = f[:, 1:2]
    f2 = f[:, 2:3]
    c0 = f0 * rep(a) + f1 * rep(b * cosv[:, 2:3]) + f2 * rep(c * cx)
    c1 = f1 * rep(b * sing) + f2 * rep(c * cy)
    c2 = f2 * rep(c * cz)
    zeros = jnp.zeros((N_ATOMS, SPH - 3), _F32)
    cart_ref[...] = jnp.concatenate([c0, c1, c2, zeros], axis=1)

    ty = ty_ref[...]
    lanes = lax.broadcasted_iota(jnp.int32, (N_ATOMS, SPH), 1)
    onehot = (lanes == ty).astype(_F32)
    embg = jnp.dot(onehot, emb_ref[...], preferred_element_type=_F32)
    zp = jnp.dot(z_ref[...], zproj_ref[...], preferred_element_type=_F32)
    zrep = jnp.broadcast_to(zp[:, None, :], (N_CRYST, ATOMS_PER, SPH)
                            ).reshape(N_ATOMS, SPH)
    x0 = embg + zrep
    x0_ref[...] = x0
    h = jnp.dot(x0, wab_ref[...], preferred_element_type=_F32)
    ha_ref[...] = h[:, :HID]
    hb_ref[...] = h[:, HID:]


def _pre_call(frac, lengths, angles, z, z_proj, emb_pad, ty, wab0):
    return pl.pallas_call(
        _pre_body,
        out_shape=[
            jax.ShapeDtypeStruct((N_ATOMS, SPH), _F32),
            jax.ShapeDtypeStruct((N_ATOMS, SPH), _F32),
            jax.ShapeDtypeStruct((N_ATOMS, HID), _F32),
            jax.ShapeDtypeStruct((N_ATOMS, HID), _F32),
        ],
    )(frac, lengths, angles, z, z_proj, emb_pad, ty, wab0)


_BE = 512


def _rbf_body(vec_ref, w_ref, b_ref, dirn_ref, *r_refs):
    v = vec_ref[...]
    d2 = jnp.sum(v * v, axis=1, keepdims=True)
    dist = jnp.sqrt(d2)
    dirn_ref[...] = v * (1.0 / (dist + 1e-8))
    k = lax.broadcasted_iota(jnp.int32, (_BE, NRBF), 1).astype(_F32)
    width = CUTOFF / (NRBF - 1)
    t = (dist - k * width) * (1.0 / width)
    rbf = jnp.exp(-0.5 * t * t)
    r_all = jnp.dot(rbf, w_ref[...], preferred_element_type=_F32) + b_ref[...]
    for j in range(NINT + 1):
        r_refs[j][...] = r_all[:, j * HID:(j + 1) * HID]


def _rbf_call(vec, wc_all, b_all):
    nout = NINT + 1
    return pl.pallas_call(
        _rbf_body,
        grid=(E // _BE,),
        in_specs=[
            pl.BlockSpec((_BE, L), lambda i: (i, 0)),
            pl.BlockSpec((SPH, nout * HID), lambda i: (0, 0)),
            pl.BlockSpec((1, nout * HID), lambda i: (0, 0)),
        ],
        out_specs=[pl.BlockSpec((_BE, L), lambda i: (i, 0))] +
                  [pl.BlockSpec((_BE, HID), lambda i: (i, 0))] * nout,
        out_shape=[jax.ShapeDtypeStruct((E, L), _F32)] +
                  [jax.ShapeDtypeStruct((E, HID), _F32)] * nout,
    )(vec, wc_all, b_all)


def _upd_body(x_ref, p_ref, cnt_ref, w2_ref, b2_ref, wab_ref,
              xo_ref, ha_ref, hb_ref):
    s = p_ref[...]
    pre = jnp.dot(s, w2_ref[...], preferred_element_type=_F32) \
        + cnt_ref[...] * b2_ref[...]
    xn = x_ref[...] + jax.nn.silu(pre)
    xo_ref[...] = xn
    h = jnp.dot(xn, wab_ref[...], preferred_element_type=_F32)
    ha_ref[...] = h[:, :HID]
    hb_ref[...] = h[:, HID:]


def _upd_call(x, p, cnt, w2, b2row, wab):
    return pl.pallas_call(
        _upd_body,
        out_shape=[
            jax.ShapeDtypeStruct((N_ATOMS, SPH), _F32),
            jax.ShapeDtypeStruct((N_ATOMS, HID), _F32),
            jax.ShapeDtypeStruct((N_ATOMS, HID), _F32),
        ],
    )(x, p, cnt, w2, b2row, wab)


_BH = 1024


def _headw_body(sf_ref, dirn_ref, wf2_ref, bf2_ref, w_ref):
    s = sf_ref[...]
    mf = jnp.sum(s * wf2_ref[...], axis=1, keepdims=True) + bf2_ref[...]
    d = jnp.concatenate([dirn_ref[...], jnp.zeros((_BH, SPH - L), _F32)],
                        axis=1)
    w_ref[...] = mf * d


def _headw_call(sf, dirn, wf2row, bf2r):
    return pl.pallas_call(
        _headw_body,
        grid=(E // _BH,),
        in_specs=[
            pl.BlockSpec((_BH, HID), lambda i: (i, 0)),
            pl.BlockSpec((_BH, L), lambda i: (i, 0)),
            pl.BlockSpec((1, HID), lambda i: (0, 0)),
            pl.BlockSpec((1, 1), lambda i: (0, 0)),
        ],
        out_specs=pl.BlockSpec((_BH, SPH), lambda i: (i, 0)),
        out_shape=jax.ShapeDtypeStruct((E, SPH), _F32),
    )(sf, dirn, wf2row, bf2r)


def _fin_body(p3_ref, x_ref, wa_ref, ba_ref, cart_ref, logit_ref):
    cart_ref[...] = p3_ref[:, 0:3]
    logit_ref[...] = jnp.dot(x_ref[...], wa_ref[...],
                             preferred_element_type=_F32) + ba_ref[...]


def _fin_call(p3, x, wa, bar):
    return pl.pallas_call(
        _fin_body,
        out_shape=[
            jax.ShapeDtypeStruct((N_ATOMS, 3), _F32),
            jax.ShapeDtypeStruct((N_ATOMS, MAX_ATOM), _F32),
        ],
    )(p3, x, wa, bar)


_MESH = plsc.VectorSubcoreMesh(core_axis_name="c", subcore_axis_name="s")


def _worker_id():
    return lax.axis_index("s") * NC + lax.axis_index("c")


def _sc_vec_body(cart, src3, dst3, vec,
                 idx_sc, idx_dc, buf_s, buf_d, buf_v, sem_s, sem_d):
    cid = lax.axis_index("c")
    sid = lax.axis_index("s")
    wid = sid * NC + cid
    for j in range(NCHUNK):
        ebase = wid * EPW + j * CHUNK
        pltpu.sync_copy(src3.at[wid, j], idx_sc)
        pltpu.sync_copy(dst3.at[wid, j], idx_dc)
        ca = pltpu.async_copy(cart.at[idx_sc], buf_s, sem_s)
        cb = pltpu.async_copy(cart.at[idx_dc], buf_d, sem_d)
        ca.wait()
        cb.wait()

        def vrow(e, c2):
            sl = pl.ds(0, L)
            buf_v[e, sl] = buf_d[e, sl] - buf_s[e, sl]
            return c2

        lax.fori_loop(0, CHUNK, vrow, 0)
        pltpu.sync_copy(buf_v, vec.at[pl.ds(ebase, CHUNK)])


_sc_vec = pl.kernel(
    _sc_vec_body,
    out_type=jax.ShapeDtypeStruct((E, L), _F32),
    mesh=_MESH,
    scratch_types=[
        pltpu.VMEM((CHUNK,), jnp.int32),
        pltpu.VMEM((CHUNK,), jnp.int32),
        pltpu.VMEM((CHUNK, SPH), _F32),
        pltpu.VMEM((CHUNK, SPH), _F32),
        pltpu.VMEM((CHUNK, L), _F32),
        pltpu.SemaphoreType.DMA,
        pltpu.SemaphoreType.DMA,
    ],
)


def _silu_rows(buf_a, buf_b, buf_r):
    def row(e, carry):
        for cc in range(HID // L):
            sl = pl.ds(cc * L, L)
            v = buf_a[e, sl] + buf_b[e, sl] + buf_r[e, sl]
            buf_a[e, sl] = v / (1.0 + jnp.exp(-v))
        return carry

    lax.fori_loop(0, CHUNK, row, 0)


def _sc_head_body(ha, hb, r, src3, dst3, sf,
                  idx_sc, idx_dc, buf_a, buf_b, buf_r, sem_a, sem_b, sem_r):
    cid = lax.axis_index("c")
    sid = lax.axis_index("s")
    wid = sid * NC + cid
    for j in range(NCHUNK):
        ebase = wid * EPW + j * CHUNK
        pltpu.sync_copy(src3.at[wid, j], idx_sc)
        pltpu.sync_copy(dst3.at[wid, j], idx_dc)
        ca = pltpu.async_copy(ha.at[idx_sc], buf_a, sem_a)
        cb = pltpu.async_copy(hb.at[idx_dc], buf_b, sem_b)
        cr = pltpu.async_copy(r.at[pl.ds(ebase, CHUNK)], buf_r, sem_r)
        ca.wait()
        cb.wait()
        cr.wait()
        _silu_rows(buf_a, buf_b, buf_r)
        pltpu.sync_copy(buf_a, sf.at[pl.ds(ebase, CHUNK)])


_sc_head = pl.kernel(
    _sc_head_body,
    out_type=jax.ShapeDtypeStruct((E, HID), _F32),
    mesh=_MESH,
    scratch_types=[
        pltpu.VMEM((CHUNK,), jnp.int32),
        pltpu.VMEM((CHUNK,), jnp.int32),
        pltpu.VMEM((CHUNK, HID), _F32),
        pltpu.VMEM((CHUNK, HID), _F32),
        pltpu.VMEM((CHUNK, HID), _F32),
        pltpu.SemaphoreType.DMA,
        pltpu.SemaphoreType.DMA,
        pltpu.SemaphoreType.DMA,
    ],
)


def _make_agg_body(ncols):
    ngrp = ncols // L

    def body(s_rows, eid_t, nloc_t, out, idx_e, nl_v, gbuf, acc, sem):
        cid = lax.axis_index("c")
        sid = lax.axis_index("s")
        wid = sid * NC + cid
        zero = jnp.zeros((L,), _F32)

        def zrow(e, c):
            for cc in range(ngrp):
                acc[e, pl.ds(cc * L, L)] = zero
            return c

        lax.fori_loop(0, NPT + 1, zrow, 0)

        def block(b, c0):
            pltpu.sync_copy(eid_t.at[wid, pl.ds(b * BB, BB)], idx_e)
            pltpu.sync_copy(nloc_t.at[wid, pl.ds(b * (BB // L), BB // L)],
                            nl_v)
            pltpu.async_copy(s_rows.at[idx_e], gbuf, sem).wait()

            def grp(g, c1):
                nlg = nl_v[g, :]
                for rr in range(L):
                    nl = nlg[rr]
                    r = g * L + rr
                    for cc in range(ngrp):
                        sl = pl.ds(cc * L, L)
                        acc[nl, sl] = acc[nl, sl] + gbuf[r, sl]
                return c1

            lax.fori_loop(0, BB // L, grp, 0)
            return c0

        lax.fori_loop(0, CAPT // BB, block, 0)
        pltpu.sync_copy(acc.at[pl.ds(0, NPT)],
                        out.at[pl.ds(wid * NPT, NPT)])

    return body


_sc_agg = pl.kernel(
    _make_agg_body(HID),
    out_type=jax.ShapeDtypeStruct((N_ATOMS, HID), _F32),
    mesh=_MESH,
    scratch_types=[
        pltpu.VMEM((BB,), jnp.int32),
        pltpu.VMEM((BB // L, L), jnp.int32),
        pltpu.VMEM((BB, HID), _F32),
        pltpu.VMEM((NPT + 1, HID), _F32),
        pltpu.SemaphoreType.DMA,
    ],
)

_sc_agg128 = pl.kernel(
    _make_agg_body(SPH),
    out_type=jax.ShapeDtypeStruct((N_ATOMS, SPH), _F32),
    mesh=_MESH,
    scratch_types=[
        pltpu.VMEM((BB,), jnp.int32),
        pltpu.VMEM((BB // L, L), jnp.int32),
        pltpu.VMEM((BB, SPH), _F32),
        pltpu.VMEM((NPT + 1, SPH), _F32),
        pltpu.SemaphoreType.DMA,
    ],
)


def kernel(z, pred_frac_coords, pred_atom_types, num_atoms, lengths, angles,
           edge_index, emb, z_proj, W1, b1, W2, b2, Wf1, bf1, Wf2, bf2,
           Wa, ba):
    del num_atoms
    i32 = jnp.int32

    w1ab = [jnp.concatenate([W1[i, :SPH, :], W1[i, SPH:2 * SPH, :]], axis=1)
            for i in range(NINT)]
    wfab = jnp.concatenate([Wf1[:SPH, :], Wf1[SPH:2 * SPH, :]], axis=1)
    wc_all = jnp.concatenate([W1[i, 2 * SPH:, :] for i in range(NINT)]
                             + [Wf1[2 * SPH:, :]], axis=1)
    b_all = jnp.concatenate([b1.reshape(-1), bf1]).reshape(1, (NINT + 1) * HID)
    emb_pad = jnp.pad(emb, ((0, SPH - MAX_ATOM), (0, 0)))
    ty = pred_atom_types.astype(i32).reshape(N_ATOMS, 1)
    srcI = edge_index[0].astype(i32)
    dstI = edge_index[1].astype(i32)
    src3 = srcI.reshape(NW, NCHUNK, CHUNK)
    dst3 = dstI.reshape(NW, NCHUNK, CHUNK)
    wf2row = Wf2.reshape(1, HID)
    bf2r = bf2.reshape(1, 1)
    bar = ba.reshape(1, MAX_ATOM)

    order = jnp.argsort(dstI).astype(i32)
    sdst = jnp.take(dstI, order)
    cnt = jnp.zeros((N_ATOMS,), i32).at[dstI].add(1)
    cum = jnp.cumsum(cnt)
    tile_start = jnp.concatenate([jnp.zeros((1,), i32),
                                  cum[:-1].astype(i32)])[::NPT]
    tile_of = sdst // NPT
    pos = jnp.arange(E, dtype=i32) - jnp.take(tile_start, tile_of)
    flat = jnp.where(pos < CAPT, tile_of * CAPT + pos, NW * CAPT)
    eid_t = jnp.zeros((NW * CAPT + 1,), i32).at[flat].set(order)
    nloc_t = jnp.full((NW * CAPT + 1,), NPT, i32).at[flat].set(sdst % NPT)
    eid_t = eid_t[:NW * CAPT].reshape(NW, CAPT)
    nloc_t = nloc_t[:NW * CAPT].reshape(NW, CAPT // L, L)
    cnt_f = cnt.astype(_F32).reshape(N_ATOMS, 1)

    cart16, x, ha, hb = _pre_call(pred_frac_coords, lengths, angles, z,
                                  z_proj, emb_pad, ty, w1ab[0])
    vec = _sc_vec(cart16, src3, dst3)
    rbf_outs = _rbf_call(vec, wc_all, b_all)
    dirn = rbf_outs[0]
    r_list = rbf_outs[1:]

    for i in range(NINT):
        s_edges = _sc_head(ha, hb, r_list[i], src3, dst3)
        p = _sc_agg(s_edges, eid_t, nloc_t)
        wab_next = w1ab[i + 1] if i + 1 < NINT else wfab
        x, ha, hb = _upd_call(x, p, cnt_f, W2[i], b2[i].reshape(1, SPH),
                              wab_next)

    sf = _sc_head(ha, hb, r_list[NINT], src3, dst3)
    w_edges = _headw_call(sf, dirn, wf2row, bf2r)
    p3 = _sc_agg128(w_edges, eid_t, nloc_t)
    pred_cart_coord_diff, pred_atom_logits = _fin_call(p3, x, Wa, bar)
    return (pred_cart_coord_diff, pred_atom_logits)
